# Initial kernel scaffold; baseline (speedup 1.0000x reference)
import jax
import jax.numpy as jnp
from jax.experimental import pallas as pl


def kernel(x, edge_index, batch, climber, W1, b1, W2, b2, We, be, Wc1, bc1, Wc2, bc2):
    raise NotImplementedError("write your pallas kernel here")


# speedup vs baseline: 6.2832x; 6.2832x over previous
import functools

import jax
import jax.numpy as jnp
from jax import lax
from jax.experimental import pallas as pl
from jax.experimental.pallas import tpu as pltpu
from jax.experimental.pallas import tpu_sc as plsc

N = 10000
NP = 10240
E = 160000
EP = 161792
D_IN = 256
HID = 512
OUT = 4
G = 16

NSL = 4
SL = 128
RPT = NP // 16
CH = 128
NCH = (EP // 16) // CH
DEG_CH = 64
DEG_NCH = (EP // 32) // DEG_CH

def _deg_body(dst_hbm, out_hbm, idx_v, ones_v, zrow_v, shared):
    c = lax.axis_index("c")
    s = lax.axis_index("s")
    for j in range(DEG_CH // 16):
        ones_v[pl.ds(j * 16, 16)] = jnp.ones((16,), jnp.float32)
    for j in range(RPT // 16):
        zrow_v[pl.ds(j * 16, 16)] = jnp.zeros((16,), jnp.float32)
    pltpu.sync_copy(zrow_v, shared.at[pl.ds(s * RPT, RPT)])
    plsc.subcore_barrier()

    half = EP // 2

    def body(k, carry):
        base = c * half + s * (half // 16) + k * DEG_CH
        pltpu.sync_copy(dst_hbm.at[pl.ds(base, DEG_CH)], idx_v)
        pltpu.sync_copy(ones_v, shared.at[idx_v], add=True)
        return carry

    lax.fori_loop(0, DEG_NCH, body, 0)
    plsc.subcore_barrier()
    pltpu.sync_copy(shared.at[pl.ds(s * RPT, RPT)],
                    out_hbm.at[pl.ds(c * NP + s * RPT, RPT)])


def _agg_body(hs_hbm, src_hbm, dst_hbm, out_hbm,
              src_v, dst_v, gidx_v, rows_v, zrow_v, shared, sem):
    c = lax.axis_index("c")
    s = lax.axis_index("s")

    def zbody(i, carry):
        for j in range(SL // 16):
            zrow_v[i, pl.ds(j * 16, 16)] = jnp.zeros((16,), jnp.float32)
        return carry

    lax.fori_loop(0, CH, zbody, 0)

    per_tile = EP // 16
    for p in range(2):
        sl = p * 2 + c
        for j in range(RPT // CH):
            pltpu.sync_copy(zrow_v, shared.at[pl.ds(s * RPT + j * CH, CH)])
        plsc.subcore_barrier()

        def body(k, carry):
            base = s * per_tile + k * CH
            pltpu.sync_copy(src_hbm.at[pl.ds(base, CH)], src_v)
            pltpu.sync_copy(dst_hbm.at[pl.ds(base, CH)], dst_v)
            off = sl * NP
            for j in range(CH // 16):
                gidx_v[pl.ds(j * 16, 16)] = src_v[pl.ds(j * 16, 16)] + off
            pltpu.async_copy(hs_hbm.at[gidx_v], rows_v, sem).wait()
            pltpu.sync_copy(rows_v, shared.at[dst_v], add=True)
            return carry

        lax.fori_loop(0, NCH, body, 0)
        plsc.subcore_barrier()
        pltpu.sync_copy(shared.at[pl.ds(s * RPT, RPT)],
                        out_hbm.at[pl.ds(sl * NP + s * RPT, RPT)])
        plsc.subcore_barrier()


@functools.lru_cache(maxsize=None)
def _build_sc_kernels():
    mesh = plsc.VectorSubcoreMesh(core_axis_name="c", subcore_axis_name="s")
    deg_k = pl.kernel(
        _deg_body,
        mesh=mesh,
        out_type=jax.ShapeDtypeStruct((2 * NP,), jnp.float32),
        scratch_types=[
            pltpu.VMEM((DEG_CH,), jnp.int32),
            pltpu.VMEM((DEG_CH,), jnp.float32),
            pltpu.VMEM((RPT,), jnp.float32),
            pltpu.VMEM_SHARED((NP,), jnp.float32),
        ],
    )
    agg_k = pl.kernel(
        _agg_body,
        mesh=mesh,
        out_type=jax.ShapeDtypeStruct((NSL * NP, SL), jnp.float32),
        scratch_types=[
            pltpu.VMEM((CH,), jnp.int32),
            pltpu.VMEM((CH,), jnp.int32),
            pltpu.VMEM((CH,), jnp.int32),
            pltpu.VMEM((CH, SL), jnp.float32),
            pltpu.VMEM((CH, SL), jnp.float32),
            pltpu.VMEM_SHARED((NP, SL), jnp.float32),
            pltpu.SemaphoreType.DMA,
        ],
    )
    return deg_k, agg_k


def _deg_sc(dstp):
    return _build_sc_kernels()[0](dstp)


def _agg_sc(hs_flat, srcp, dstp):
    return _build_sc_kernels()[1](hs_flat, srcp, dstp)


_RB = 512
_NRB = NP // _RB


def _mm1_body(x_ref, w_ref, deg_ref, hs_ref, dinv_ref):
    d = lax.rsqrt(1.0 + deg_ref[:, 0:1] + deg_ref[:, 1:2])
    dinv_ref[...] = d
    acc = jnp.dot(x_ref[...], w_ref[...], preferred_element_type=jnp.float32)
    hs_ref[...] = (d * acc)[None]


def _mm1(xp, w1, deg_t):
    return pl.pallas_call(
        _mm1_body,
        grid=(_NRB, NSL),
        in_specs=[
            pl.BlockSpec((_RB, D_IN), lambda i, j: (i, 0)),
            pl.BlockSpec((D_IN, SL), lambda i, j: (0, j)),
            pl.BlockSpec((_RB, 2), lambda i, j: (i, 0)),
        ],
        out_specs=[
            pl.BlockSpec((1, _RB, SL), lambda i, j: (j, i, 0)),
            pl.BlockSpec((_RB, 1), lambda i, j: (i, 0)),
        ],
        out_shape=[
            jax.ShapeDtypeStruct((NSL, NP, SL), jnp.float32),
            jax.ShapeDtypeStruct((NP, 1), jnp.float32),
        ],
    )(xp, w1, deg_t)


def _mm2_body(s_ref, hs_ref, dinv_ref, b_ref, w_ref, out_ref):
    k = pl.program_id(2)
    d = dinv_ref[...]
    h = jax.nn.relu(d * (s_ref[0] + hs_ref[0]) + b_ref[0])
    @pl.when(k == 0)
    def _():
        out_ref[...] = jnp.zeros_like(out_ref)
    out_ref[...] += jnp.dot(h, w_ref[...],
                            preferred_element_type=jnp.float32)[None]
    @pl.when(k == NSL - 1)
    def _():
        out_ref[...] *= d[None]


def _mm2(s1, hs1, dinv, b1r, w2):
    return pl.pallas_call(
        _mm2_body,
        grid=(_NRB, NSL, NSL),
        in_specs=[
            pl.BlockSpec((1, _RB, SL), lambda i, j, k: (k, i, 0)),
            pl.BlockSpec((1, _RB, SL), lambda i, j, k: (k, i, 0)),
            pl.BlockSpec((_RB, 1), lambda i, j, k: (i, 0)),
            pl.BlockSpec((1, 1, SL), lambda i, j, k: (k, 0, 0)),
            pl.BlockSpec((SL, SL), lambda i, j, k: (k, j)),
        ],
        out_specs=pl.BlockSpec((1, _RB, SL), lambda i, j, k: (j, i, 0)),
        out_shape=jax.ShapeDtypeStruct((NSL, NP, SL), jnp.float32),
    )(s1, hs1, dinv, b1r, w2)


def _mm3_body(cl_ref, we_ref, be_ref, wb_ref, bc1_ref, go_ref):
    ce = jnp.dot(cl_ref[...], we_ref[...],
                 preferred_element_type=jnp.float32) + be_ref[...]
    go_ref[...] = jnp.dot(ce, wb_ref[...],
                          preferred_element_type=jnp.float32) + bc1_ref[...]


def _mm3(climber, we, be_r, wc1_bot, bc1_r):
    return pl.pallas_call(
        _mm3_body,
        out_shape=jax.ShapeDtypeStruct((G, HID), jnp.float32),
    )(climber, we, be_r, wc1_bot, bc1_r)


def _mm4_body(s_ref, hs_ref, dinv_ref, b_ref, wt_ref, go_ref, batch_ref,
              wc2_ref, bc2_ref, out_ref):
    d = dinv_ref[...]
    z1 = jnp.zeros((_RB, HID), jnp.float32)
    for k in range(NSL):
        h = jax.nn.relu(d * (s_ref[k] + hs_ref[k]) + b_ref[k])
        z1 += jnp.dot(h, wt_ref[pl.ds(k * SL, SL), :],
                      preferred_element_type=jnp.float32)
    iota = lax.broadcasted_iota(jnp.int32, (_RB, 128), 1)
    onehot = (batch_ref[...] == iota).astype(jnp.float32)
    z1 += jnp.dot(onehot, go_ref[...], preferred_element_type=jnp.float32)
    z1 = jax.nn.relu(z1)
    out_ref[...] = jnp.dot(z1, wc2_ref[...],
                           preferred_element_type=jnp.float32) + bc2_ref[...]


def _mm4(s2, hs2, dinv, b2r, wc1_top, go_pad, batch_p, wc2_pad, bc2_pad):
    return pl.pallas_call(
        _mm4_body,
        grid=(_NRB,),
        in_specs=[
            pl.BlockSpec((NSL, _RB, SL), lambda i: (0, i, 0)),
            pl.BlockSpec((NSL, _RB, SL), lambda i: (0, i, 0)),
            pl.BlockSpec((_RB, 1), lambda i: (i, 0)),
            pl.BlockSpec((NSL, 1, SL), lambda i: (0, 0, 0)),
            pl.BlockSpec((HID, HID), lambda i: (0, 0)),
            pl.BlockSpec((128, HID), lambda i: (0, 0)),
            pl.BlockSpec((_RB, 1), lambda i: (i, 0)),
            pl.BlockSpec((HID, 128), lambda i: (0, 0)),
            pl.BlockSpec((1, 128), lambda i: (0, 0)),
        ],
        out_specs=pl.BlockSpec((_RB, 128), lambda i: (i, 0)),
        out_shape=jax.ShapeDtypeStruct((NP, 128), jnp.float32),
    )(s2, hs2, dinv, b2r, wc1_top, go_pad, batch_p, wc2_pad, bc2_pad)


def kernel(x, edge_index, batch, climber, W1, b1, W2, b2, We, be,
           Wc1, bc1, Wc2, bc2):
    npad = EP - E
    pad_iota = jnp.arange(npad, dtype=jnp.int32) % 8
    srcp = jnp.concatenate([edge_index[0], pad_iota])
    dstp = jnp.concatenate([edge_index[1], N + pad_iota])

    deg_flat = _deg_sc(dstp)
    deg_t = deg_flat.reshape(2, NP).T

    xp = jnp.pad(x, ((0, NP - N), (0, 0)))
    hs1, dinv = _mm1(xp, W1, deg_t)

    s1 = _agg_sc(hs1.reshape(NSL * NP, SL), srcp, dstp).reshape(NSL, NP, SL)

    b1r = b1.reshape(NSL, 1, SL)
    hs2 = _mm2(s1, hs1, dinv, b1r, W2)

    s2 = _agg_sc(hs2.reshape(NSL * NP, SL), srcp, dstp).reshape(NSL, NP, SL)

    go = _mm3(climber, We, be.reshape(1, HID), Wc1[HID:], bc1.reshape(1, HID))
    go_pad = jnp.pad(go, ((0, 128 - G), (0, 0)))
    batch_p = jnp.pad(batch, (0, NP - N)).reshape(NP, 1)
    wc2_pad = jnp.pad(Wc2, ((0, 0), (0, 128 - OUT)))
    bc2_pad = jnp.pad(bc2, (0, 128 - OUT)).reshape(1, 128)
    b2r = b2.reshape(NSL, 1, SL)

    out = _mm4(s2, hs2, dinv, b2r, Wc1[:HID], go_pad, batch_p,
               wc2_pad, bc2_pad)
    return out[:N, :OUT]


# speedup vs baseline: 9.7639x; 1.5540x over previous
import functools

import jax
import jax.numpy as jnp
from jax import lax
from jax.experimental import pallas as pl
from jax.experimental.pallas import tpu as pltpu
from jax.experimental.pallas import tpu_sc as plsc

N = 10000
NP = 10240
E = 160000
EP = 163840
D_IN = 256
HID = 512
OUT = 4
G = 16

NSL = 4
SL = 128
RPT = NP // 16
CH = 128
NCH = (EP // 16) // CH
NQ = NCH // 4
DEG_NCH = (EP // 32) // CH

def _deg_body(dst_hbm, out_hbm, idx_v, ones_v, zrow_v, shared):
    c = lax.axis_index("c")
    s = lax.axis_index("s")
    w = s * 2 + c
    for j in range(CH // 16):
        ones_v[pl.ds(j * 16, 16)] = jnp.ones((16,), jnp.float32)
    for j in range(RPT // 16):
        zrow_v[pl.ds(j * 16, 16)] = jnp.zeros((16,), jnp.float32)
    pltpu.sync_copy(zrow_v, shared.at[pl.ds(s * RPT, RPT)])
    pltpu.sync_copy(dst_hbm.at[w], idx_v)
    plsc.subcore_barrier()

    def body(k, carry):
        pltpu.sync_copy(ones_v, shared.at[idx_v.at[k]], add=True)
        return carry

    lax.fori_loop(0, DEG_NCH, body, 0)
    plsc.subcore_barrier()
    pltpu.sync_copy(shared.at[pl.ds(s * RPT, RPT)],
                    out_hbm.at[pl.ds(c * NP + s * RPT, RPT)])


def _agg_body(hs_hbm, src_hbm, dst_hbm, out_hbm,
              gidx0, gidx1, didx0, didx1, rows0, rows1, shared,
              gsem0, gsem1, isem0, isem1):
    c = lax.axis_index("c")
    s = lax.axis_index("s")
    gidx = (gidx0, gidx1)
    didx = (didx0, didx1)
    rows = (rows0, rows1)
    gsems = (gsem0, gsem1)
    isems = (isem0, isem1)
    ebase = s * (EP // 16)

    def start_idx(k, b):
        pltpu.async_copy(src_hbm.at[pl.ds(ebase + k * CH, CH)], gidx[b],
                         isems[b])
        pltpu.async_copy(dst_hbm.at[pl.ds(ebase + k * CH, CH)], didx[b],
                         isems[b])

    def wait_idx(b):
        pltpu.make_async_copy(src_hbm.at[pl.ds(0, CH)], gidx[b],
                              isems[b]).wait()
        pltpu.make_async_copy(dst_hbm.at[pl.ds(0, CH)], didx[b],
                              isems[b]).wait()

    def start_gather(b, off):
        for j in range(CH // 16):
            gidx[b][pl.ds(j * 16, 16)] += off
        pltpu.async_copy(hs_hbm.at[gidx[b]], rows[b], gsems[b])

    def wait_gather(b):
        pltpu.make_async_copy(hs_hbm.at[pl.ds(0, CH)], rows[b],
                              gsems[b]).wait()

    for p in range(2):
        sl = p * 2 + c
        off = sl * NP
        def zbody(i, carry):
            for j in range(SL // 16):
                rows0[i, pl.ds(j * 16, 16)] = jnp.zeros((16,), jnp.float32)
            return carry
        lax.fori_loop(0, CH, zbody, 0)
        for j in range(RPT // CH):
            pltpu.sync_copy(rows0, shared.at[pl.ds(s * RPT + j * CH, CH)])
        plsc.subcore_barrier()

        start_idx(0, 0)
        start_idx(1, 1)
        wait_idx(0)
        start_gather(0, off)

        def pair(t, carry):
            for b in range(2):
                k = 2 * t + b
                b2 = 1 - b
                wait_gather(b)
                if b == 0:
                    wait_idx(b2)
                    start_gather(b2, off)
                else:
                    @pl.when(t < NCH // 2 - 1)
                    def _():
                        wait_idx(b2)
                        start_gather(b2, off)
                pltpu.sync_copy(rows[b], shared.at[didx[b]], add=True)
                @pl.when(k + 2 < NCH)
                def _():
                    start_idx(k + 2, b)
            return carry

        lax.fori_loop(0, NCH // 2, pair, 0)
        plsc.subcore_barrier()
        pltpu.sync_copy(shared.at[pl.ds(s * RPT, RPT)],
                        out_hbm.at[pl.ds(sl * NP + s * RPT, RPT)])
        plsc.subcore_barrier()


@functools.lru_cache(maxsize=None)
def _build_sc_kernels():
    mesh = plsc.VectorSubcoreMesh(core_axis_name="c", subcore_axis_name="s")
    deg_k = pl.kernel(
        _deg_body,
        mesh=mesh,
        out_type=jax.ShapeDtypeStruct((2 * NP,), jnp.float32),
        scratch_types=[
            pltpu.VMEM((DEG_NCH, CH), jnp.int32),
            pltpu.VMEM((CH,), jnp.float32),
            pltpu.VMEM((RPT,), jnp.float32),
            pltpu.VMEM_SHARED((NP,), jnp.float32),
        ],
    )
    agg_k = pl.kernel(
        _agg_body,
        mesh=mesh,
        out_type=jax.ShapeDtypeStruct((NSL * NP, SL), jnp.float32),
        scratch_types=[
            pltpu.VMEM((CH,), jnp.int32),
            pltpu.VMEM((CH,), jnp.int32),
            pltpu.VMEM((CH,), jnp.int32),
            pltpu.VMEM((CH,), jnp.int32),
            pltpu.VMEM((CH, SL), jnp.float32),
            pltpu.VMEM((CH, SL), jnp.float32),
            pltpu.VMEM_SHARED((NP, SL), jnp.float32),
            pltpu.SemaphoreType.DMA,
            pltpu.SemaphoreType.DMA,
            pltpu.SemaphoreType.DMA,
            pltpu.SemaphoreType.DMA,
        ],
    )
    return deg_k, agg_k


def _deg_sc(dstp):
    return _build_sc_kernels()[0](dstp)


def _agg_sc(hs_flat, srcp, dstp):
    return _build_sc_kernels()[1](hs_flat, srcp, dstp)


_RB = 512
_NRB = NP // _RB


def _mm1_body(x_ref, w_ref, deg_ref, hs_ref, dinv_ref):
    d = lax.rsqrt(1.0 + deg_ref[:, 0:1] + deg_ref[:, 1:2])
    dinv_ref[...] = d
    acc = jnp.dot(x_ref[...], w_ref[...], preferred_element_type=jnp.float32)
    hs_ref[...] = (d * acc)[None]


def _mm1(xp, w1, deg_t):
    return pl.pallas_call(
        _mm1_body,
        grid=(_NRB, NSL),
        in_specs=[
            pl.BlockSpec((_RB, D_IN), lambda i, j: (i, 0)),
            pl.BlockSpec((D_IN, SL), lambda i, j: (0, j)),
            pl.BlockSpec((_RB, 2), lambda i, j: (i, 0)),
        ],
        out_specs=[
            pl.BlockSpec((1, _RB, SL), lambda i, j: (j, i, 0)),
            pl.BlockSpec((_RB, 1), lambda i, j: (i, 0)),
        ],
        out_shape=[
            jax.ShapeDtypeStruct((NSL, NP, SL), jnp.float32),
            jax.ShapeDtypeStruct((NP, 1), jnp.float32),
        ],
    )(xp, w1, deg_t)


def _mm2_body(s_ref, hs_ref, dinv_ref, b_ref, w_ref, out_ref):
    k = pl.program_id(2)
    d = dinv_ref[...]
    h = jax.nn.relu(d * (s_ref[0] + hs_ref[0]) + b_ref[0])
    @pl.when(k == 0)
    def _():
        out_ref[...] = jnp.zeros_like(out_ref)
    out_ref[...] += jnp.dot(h, w_ref[...],
                            preferred_element_type=jnp.float32)[None]
    @pl.when(k == NSL - 1)
    def _():
        out_ref[...] *= d[None]


def _mm2(s1, hs1, dinv, b1r, w2):
    return pl.pallas_call(
        _mm2_body,
        grid=(_NRB, NSL, NSL),
        in_specs=[
            pl.BlockSpec((1, _RB, SL), lambda i, j, k: (k, i, 0)),
            pl.BlockSpec((1, _RB, SL), lambda i, j, k: (k, i, 0)),
            pl.BlockSpec((_RB, 1), lambda i, j, k: (i, 0)),
            pl.BlockSpec((1, 1, SL), lambda i, j, k: (k, 0, 0)),
            pl.BlockSpec((SL, SL), lambda i, j, k: (k, j)),
        ],
        out_specs=pl.BlockSpec((1, _RB, SL), lambda i, j, k: (j, i, 0)),
        out_shape=jax.ShapeDtypeStruct((NSL, NP, SL), jnp.float32),
    )(s1, hs1, dinv, b1r, w2)


def _mm3_body(cl_ref, we_ref, be_ref, wb_ref, bc1_ref, go_ref):
    ce = jnp.dot(cl_ref[...], we_ref[...],
                 preferred_element_type=jnp.float32) + be_ref[...]
    go_ref[...] = jnp.dot(ce, wb_ref[...],
                          preferred_element_type=jnp.float32) + bc1_ref[...]


def _mm3(climber, we, be_r, wc1_bot, bc1_r):
    return pl.pallas_call(
        _mm3_body,
        out_shape=jax.ShapeDtypeStruct((G, HID), jnp.float32),
    )(climber, we, be_r, wc1_bot, bc1_r)


def _mm4_body(s_ref, hs_ref, dinv_ref, b_ref, wt_ref, go_ref, batch_ref,
              wc2_ref, bc2_ref, out_ref):
    d = dinv_ref[...]
    z1 = jnp.zeros((_RB, HID), jnp.float32)
    for k in range(NSL):
        h = jax.nn.relu(d * (s_ref[k] + hs_ref[k]) + b_ref[k])
        z1 += jnp.dot(h, wt_ref[pl.ds(k * SL, SL), :],
                      preferred_element_type=jnp.float32)
    iota = lax.broadcasted_iota(jnp.int32, (_RB, 128), 1)
    onehot = (batch_ref[...] == iota).astype(jnp.float32)
    z1 += jnp.dot(onehot, go_ref[...], preferred_element_type=jnp.float32)
    z1 = jax.nn.relu(z1)
    out_ref[...] = jnp.dot(z1, wc2_ref[...],
                           preferred_element_type=jnp.float32) + bc2_ref[...]


def _mm4(s2, hs2, dinv, b2r, wc1_top, go_pad, batch_p, wc2_pad, bc2_pad):
    return pl.pallas_call(
        _mm4_body,
        grid=(_NRB,),
        in_specs=[
            pl.BlockSpec((NSL, _RB, SL), lambda i: (0, i, 0)),
            pl.BlockSpec((NSL, _RB, SL), lambda i: (0, i, 0)),
            pl.BlockSpec((_RB, 1), lambda i: (i, 0)),
            pl.BlockSpec((NSL, 1, SL), lambda i: (0, 0, 0)),
            pl.BlockSpec((HID, HID), lambda i: (0, 0)),
            pl.BlockSpec((128, HID), lambda i: (0, 0)),
            pl.BlockSpec((_RB, 1), lambda i: (i, 0)),
            pl.BlockSpec((HID, 128), lambda i: (0, 0)),
            pl.BlockSpec((1, 128), lambda i: (0, 0)),
        ],
        out_specs=pl.BlockSpec((_RB, 128), lambda i: (i, 0)),
        out_shape=jax.ShapeDtypeStruct((NP, 128), jnp.float32),
    )(s2, hs2, dinv, b2r, wc1_top, go_pad, batch_p, wc2_pad, bc2_pad)


def kernel(x, edge_index, batch, climber, W1, b1, W2, b2, We, be,
           Wc1, bc1, Wc2, bc2):
    npad = EP - E
    pad_iota = jnp.arange(npad, dtype=jnp.int32) % 16
    srcp = jnp.concatenate([edge_index[0], pad_iota])
    dstp = jnp.concatenate([edge_index[1], N + pad_iota])
    dst32 = dstp.reshape(32, DEG_NCH, CH)

    deg_flat = _deg_sc(dst32)
    deg_t = deg_flat.reshape(2, NP).T

    xp = jnp.pad(x, ((0, NP - N), (0, 0)))
    hs1, dinv = _mm1(xp, W1, deg_t)

    s1 = _agg_sc(hs1.reshape(NSL * NP, SL), srcp, dstp).reshape(NSL, NP, SL)

    b1r = b1.reshape(NSL, 1, SL)
    hs2 = _mm2(s1, hs1, dinv, b1r, W2)

    s2 = _agg_sc(hs2.reshape(NSL * NP, SL), srcp, dstp).reshape(NSL, NP, SL)

    go = _mm3(climber, We, be.reshape(1, HID), Wc1[HID:], bc1.reshape(1, HID))
    go_pad = jnp.pad(go, ((0, 128 - G), (0, 0)))
    batch_p = jnp.pad(batch, (0, NP - N)).reshape(NP, 1)
    wc2_pad = jnp.pad(Wc2, ((0, 0), (0, 128 - OUT)))
    bc2_pad = jnp.pad(bc2, (0, 128 - OUT)).reshape(1, 128)
    b2r = b2.reshape(NSL, 1, SL)

    out = _mm4(s2, hs2, dinv, b2r, Wc1[:HID], go_pad, batch_p,
               wc2_pad, bc2_pad)
    return out[:N, :OUT]
